# baseline (speedup 1.0000x reference)
import functools

import jax
import jax.numpy as jnp
import numpy as np
from jax.experimental import pallas as pl
from jax.experimental.pallas import tpu as pltpu

_MASK_ID = 99999
_PAD_ID = 0
_MAX_STEPS = 64
_DT = (1.0 - 1e-05) / (_MAX_STEPS + 1)

_KS0 = np.uint32(1948878966)
_KS1 = np.uint32(4237131848)
_KS2 = np.uint32(int(_KS0) ^ int(_KS1) ^ 0x1BD11BDA)
_KS = (_KS0, _KS1, _KS2)
_ROT = ((13, 15, 26, 6), (17, 29, 16, 24))

_SUB = 8
_CH = 1792


def _threefry_bits(n):
    x0 = jnp.full(n.shape, _KS0, dtype=jnp.uint32)
    x1 = n + _KS1
    for step in range(5):
        for r in _ROT[step % 2]:
            x0 = x0 + x1
            x1 = ((x1 << r) | (x1 >> (32 - r))) ^ x0
        x0 = x0 + _KS[(step + 1) % 3]
        x1 = x1 + _KS[(step + 2) % 3] + np.uint32(step + 1)
    return x0 ^ x1


def _tau_step_kernel(order_ref, nm_ref, coef_ref, xt_ref, vl_ref,
                     rate_ref, newxt_ref, *, V, W, WP):
    i = pl.program_id(0)
    xt_v = xt_ref[0]
    newxt_ref[0] = xt_v

    @pl.when(i < nm_ref[0])
    def _compute():
        n_full = WP // _CH - 1
        tail = n_full * _CH
        lane0 = jax.lax.broadcasted_iota(jnp.int32, (_SUB, _CH), 1)
        sub0 = jax.lax.broadcasted_iota(jnp.int32, (_SUB, _CH), 0)
        tvalid = lane0 < (W - tail)

        def max_body(c, m):
            return jnp.maximum(m, vl_ref[0, :, pl.ds(c * _CH, _CH)])

        m = jax.lax.fori_loop(
            0, n_full, max_body,
            jnp.full((_SUB, _CH), -jnp.inf, jnp.float32))
        x_t = vl_ref[0, :, pl.ds(tail, _CH)]
        m = jnp.maximum(m, jnp.where(tvalid, x_t, -jnp.inf))
        rowmax = jnp.max(m)

        def sum_body(c, s):
            return s + jnp.exp(vl_ref[0, :, pl.ds(c * _CH, _CH)] - rowmax)

        s = jax.lax.fori_loop(
            0, n_full, sum_body, jnp.zeros((_SUB, _CH), jnp.float32))
        s = s + jnp.where(tvalid, jnp.exp(x_t - rowmax), 0.0)
        recip = coef_ref[0] / jnp.sum(s)

        row = order_ref[i]
        base = row * V + sub0 * W

        def rng_cond(start, r_c):
            n = (base + (lane0 + start)).astype(jnp.uint32)
            bits = _threefry_bits(n)
            u = jax.lax.bitcast_convert_type(
                (bits >> 9) | np.uint32(0x3F800000),
                jnp.float32) - jnp.float32(1.0)
            return jnp.log(u) > -r_c

        def b_body(c, acc):
            start = c * _CH
            x_c = vl_ref[0, :, pl.ds(start, _CH)]
            r_c = jnp.exp(x_c - rowmax) * recip
            rate_ref[0, :, pl.ds(start, _CH)] = r_c
            return acc | rng_cond(start, r_c).astype(jnp.int32)

        acc = jax.lax.fori_loop(
            0, n_full, b_body, jnp.zeros((_SUB, _CH), jnp.int32))
        r_t = jnp.exp(x_t - rowmax) * recip
        is_last = (sub0 == (_SUB - 1)) & (lane0 == (W - 1 - tail))
        r_t = jnp.where(is_last, jnp.float32(0.0), r_t)
        rate_ref[0, :, pl.ds(tail, _CH)] = r_t
        acc = acc | (rng_cond(tail, r_t) & tvalid).astype(jnp.int32)

        @pl.when(jnp.max(acc) > 0)
        def _reveal():
            flat0 = sub0 * W + lane0

            def argmax_body(c, best):
                x_c = vl_ref[0, :, pl.ds(c * _CH, _CH)]
                return jnp.minimum(
                    best, jnp.where(x_c == rowmax, flat0 + c * _CH, V))

            best = jax.lax.fori_loop(
                0, n_full, argmax_body,
                jnp.full((_SUB, _CH), V, jnp.int32))
            best = jnp.minimum(
                best, jnp.where((x_t == rowmax) & tvalid, flat0 + tail, V))
            amax = jnp.min(best)
            new = jnp.where(xt_v == _MASK_ID, amax, xt_v)
            newxt_ref[0] = jnp.where(xt_v == _PAD_ID, _PAD_ID, new)

    @pl.when(i >= nm_ref[0])
    def _zero():
        rate_ref[0] = jnp.zeros((_SUB, WP), jnp.float32)


@jax.jit
def kernel(vocab_logits, xt, t):
    B, L, V = vocab_logits.shape
    n_rows = B * L
    W = V // _SUB
    WP = ((W + _CH - 1) // _CH) * _CH

    xtf = xt.reshape(n_rows)
    flags = xtf == _MASK_ID
    order = jnp.argsort(jnp.logical_not(flags), stable=True).astype(jnp.int32)
    n_masked = jnp.reshape(flags.sum().astype(jnp.int32), (1,))

    factor = 1.0 / (1.0 - t + 1e-4)
    coef = jnp.float32(_DT) * factor[:, None] * flags.reshape(B, L)
    coef3 = coef.astype(jnp.float32).reshape(n_rows, 1, 1)
    xt3 = xtf.reshape(n_rows, 1, 1)
    vl3 = vocab_logits.reshape(n_rows, _SUB, W)

    def gather_map(i, order_ref, nm_ref):
        return (order_ref[i], 0, 0)

    rate3, newxt3 = pl.pallas_call(
        functools.partial(_tau_step_kernel, V=V, W=W, WP=WP),
        grid_spec=pltpu.PrefetchScalarGridSpec(
            num_scalar_prefetch=2,
            grid=(n_rows,),
            in_specs=[
                pl.BlockSpec((1, 1, 1), gather_map),
                pl.BlockSpec((1, 1, 1), gather_map),
                pl.BlockSpec((1, _SUB, WP), gather_map),
            ],
            out_specs=[
                pl.BlockSpec((1, _SUB, WP), gather_map),
                pl.BlockSpec((1, 1, 1), gather_map),
            ],
        ),
        out_shape=[
            jax.ShapeDtypeStruct((n_rows, _SUB, W), jnp.float32),
            jax.ShapeDtypeStruct((n_rows, 1, 1), jnp.int32),
        ],
        compiler_params=pltpu.CompilerParams(
            dimension_semantics=("arbitrary",),
        ),
    )(order, n_masked, coef3, xt3, vl3)

    return newxt3.reshape(B, L), rate3.reshape(B, L, V)


# speedup vs baseline: 1.0444x; 1.0444x over previous
import functools

import jax
import jax.numpy as jnp
import numpy as np
from jax.experimental import pallas as pl
from jax.experimental.pallas import tpu as pltpu

_MASK_ID = 99999
_PAD_ID = 0
_MAX_STEPS = 64
_DT = (1.0 - 1e-05) / (_MAX_STEPS + 1)

_KS0 = np.uint32(1948878966)
_KS1 = np.uint32(4237131848)
_KS2 = np.uint32(int(_KS0) ^ int(_KS1) ^ 0x1BD11BDA)
_KS = (_KS0, _KS1, _KS2)
_ROT = ((13, 15, 26, 6), (17, 29, 16, 24))

_SUB = 8
_CH = 1792
_G = 8


def _threefry_bits(n):
    x0 = jnp.full(n.shape, _KS0, dtype=jnp.uint32)
    x1 = n + _KS1
    for step in range(5):
        for r in _ROT[step % 2]:
            x0 = x0 + x1
            x1 = ((x1 << r) | (x1 >> (32 - r))) ^ x0
        x0 = x0 + _KS[(step + 1) % 3]
        x1 = x1 + _KS[(step + 2) % 3] + np.uint32(step + 1)
    return x0 ^ x1


def _tau_step_kernel(order_ref, nm_ref, coef_ref, xt_ref,
                     vl_hbm, rate_hbm, newxt_ref,
                     x_buf, r_buf, in_sems, out_sems, *, V, W):
    i = pl.program_id(0)
    nprog = pl.num_programs(0)
    nm = nm_ref[0]
    slot = jax.lax.rem(i, 2)
    nslot = jax.lax.rem(i + 1, 2)

    def in_copy(prog, sl, k):
        return pltpu.make_async_copy(
            vl_hbm.at[pl.ds(order_ref[prog * _G + k], 1)],
            x_buf.at[pl.ds(sl * _G + k, 1)],
            in_sems.at[sl, k])

    def out_copy(prog, sl, k):
        return pltpu.make_async_copy(
            r_buf.at[pl.ds(sl * _G + k, 1)],
            rate_hbm.at[pl.ds(order_ref[prog * _G + k], 1)],
            out_sems.at[sl, k])

    def issue_in(prog, sl):
        for k in range(_G):
            @pl.when(prog * _G + k < nm)
            def _(k=k):
                in_copy(prog, sl, k).start()

    @pl.when(i == 0)
    def _():
        issue_in(0, 0)

    @pl.when(i + 1 < nprog)
    def _():
        issue_in(i + 1, nslot)

    @pl.when(i >= 2)
    def _():
        for k in range(_G):
            out_copy(i - 2, slot, k).wait()

    xt_vec = xt_ref[0]
    newxt_ref[0] = xt_vec
    coef_vec = coef_ref[0]
    kidx = jax.lax.broadcasted_iota(jnp.int32, (_G, 1), 0)

    tail = W - _CH
    n_full = tail // _CH + 1
    over = n_full * _CH - tail
    lane0 = jax.lax.broadcasted_iota(jnp.int32, (_SUB, _CH), 1)
    sub0 = jax.lax.broadcasted_iota(jnp.int32, (_SUB, _CH), 0)
    is_last = (sub0 == (_SUB - 1)) & (lane0 == (_CH - 1))

    for k in range(_G):
        @pl.when(i * _G + k < nm)
        def _compute(k=k):
            in_copy(i, slot, k).wait()

            def max_body(c, m):
                return jnp.maximum(m, x_buf[slot * _G + k, :, pl.ds(c * _CH, _CH)])

            m = jax.lax.fori_loop(
                0, n_full, max_body,
                jnp.full((_SUB, _CH), -jnp.inf, jnp.float32))
            x_t = x_buf[slot * _G + k, :, pl.ds(tail, _CH)]
            m = jnp.maximum(m, x_t)
            rowmax = jnp.max(m)

            def sum_body(c, s):
                x_c = x_buf[slot * _G + k, :, pl.ds(c * _CH, _CH)]
                return s + jnp.exp(x_c - rowmax)

            s = jax.lax.fori_loop(
                0, n_full, sum_body, jnp.zeros((_SUB, _CH), jnp.float32))
            s = s + jnp.where(lane0 >= over, jnp.exp(x_t - rowmax), 0.0)
            coef_k = jnp.sum(jnp.where(kidx == k, coef_vec, 0.0))
            recip = coef_k / jnp.sum(s)

            row = order_ref[i * _G + k]
            base = row * V + sub0 * W

            def rng_cond(start, r_c):
                n = (base + (lane0 + start)).astype(jnp.uint32)
                bits = _threefry_bits(n)
                u = jax.lax.bitcast_convert_type(
                    (bits >> 9) | np.uint32(0x3F800000),
                    jnp.float32) - jnp.float32(1.0)
                return jnp.log(u) > -r_c

            def b_body(c, acc):
                start = c * _CH
                x_c = x_buf[slot * _G + k, :, pl.ds(start, _CH)]
                r_c = jnp.exp(x_c - rowmax) * recip
                r_buf[slot * _G + k, :, pl.ds(start, _CH)] = r_c
                return acc | rng_cond(start, r_c).astype(jnp.int32)

            acc = jax.lax.fori_loop(
                0, n_full, b_body, jnp.zeros((_SUB, _CH), jnp.int32))
            r_t = jnp.exp(x_t - rowmax) * recip
            r_t = jnp.where(is_last, jnp.float32(0.0), r_t)
            r_buf[slot * _G + k, :, pl.ds(tail, _CH)] = r_t
            acc = acc | rng_cond(tail, r_t).astype(jnp.int32)

            @pl.when(jnp.max(acc) > 0)
            def _reveal():
                flat0 = sub0 * W + lane0

                def argmax_body(c, best):
                    x_c = x_buf[slot * _G + k, :, pl.ds(c * _CH, _CH)]
                    return jnp.minimum(
                        best, jnp.where(x_c == rowmax, flat0 + c * _CH, V))

                best = jax.lax.fori_loop(
                    0, n_full, argmax_body,
                    jnp.full((_SUB, _CH), V, jnp.int32))
                best = jnp.minimum(
                    best, jnp.where(x_t == rowmax, flat0 + tail, V))
                amax = jnp.min(best)
                hit = (kidx == k) & (xt_vec == _MASK_ID)
                newxt_ref[0] = jnp.where(hit, amax, newxt_ref[0])

        @pl.when(i * _G + k >= nm)
        def _zero(k=k):
            r_buf[slot * _G + k] = jnp.zeros((_SUB, W), jnp.float32)

    for k in range(_G):
        out_copy(i, slot, k).start()

    @pl.when(i == nprog - 1)
    def _drain():
        for k in range(_G):
            out_copy(i, slot, k).wait()
        for k in range(_G):
            out_copy(i - 1, nslot, k).wait()


@jax.jit
def kernel(vocab_logits, xt, t):
    B, L, V = vocab_logits.shape
    n_rows = B * L
    n_steps = n_rows // _G
    W = V // _SUB

    xtf = xt.reshape(n_rows)
    flags = xtf == _MASK_ID
    order = jnp.argsort(jnp.logical_not(flags), stable=True).astype(jnp.int32)
    inv = jnp.zeros((n_rows,), jnp.int32).at[order].set(
        jnp.arange(n_rows, dtype=jnp.int32))
    n_masked = jnp.reshape(flags.sum().astype(jnp.int32), (1,))

    factor = 1.0 / (1.0 - t + 1e-4)
    coef = jnp.float32(_DT) * factor[:, None] * flags.reshape(B, L)
    coefp3 = coef.astype(jnp.float32).reshape(n_rows)[order]
    coefp3 = coefp3.reshape(n_steps, _G, 1)
    xtp3 = xtf[order].reshape(n_steps, _G, 1)
    vl3 = vocab_logits.reshape(n_rows, _SUB, W)

    rate3, newxt3 = pl.pallas_call(
        functools.partial(_tau_step_kernel, V=V, W=W),
        grid_spec=pltpu.PrefetchScalarGridSpec(
            num_scalar_prefetch=2,
            grid=(n_steps,),
            in_specs=[
                pl.BlockSpec((1, _G, 1), lambda i, *_: (i, 0, 0)),
                pl.BlockSpec((1, _G, 1), lambda i, *_: (i, 0, 0)),
                pl.BlockSpec(memory_space=pl.ANY),
            ],
            out_specs=[
                pl.BlockSpec(memory_space=pl.ANY),
                pl.BlockSpec((1, _G, 1), lambda i, *_: (i, 0, 0)),
            ],
            scratch_shapes=[
                pltpu.VMEM((2 * _G, _SUB, W), jnp.float32),
                pltpu.VMEM((2 * _G, _SUB, W), jnp.float32),
                pltpu.SemaphoreType.DMA((2, _G)),
                pltpu.SemaphoreType.DMA((2, _G)),
            ],
        ),
        out_shape=[
            jax.ShapeDtypeStruct((n_rows, _SUB, W), jnp.float32),
            jax.ShapeDtypeStruct((n_steps, _G, 1), jnp.int32),
        ],
        compiler_params=pltpu.CompilerParams(
            dimension_semantics=("arbitrary",),
        ),
    )(order, n_masked, coefp3, xtp3, vl3)

    newxt = newxt3.reshape(n_rows)[inv].reshape(B, L)
    return newxt, rate3.reshape(B, L, V)


# speedup vs baseline: 1.1192x; 1.0716x over previous
import functools

import jax
import jax.numpy as jnp
import numpy as np
from jax.experimental import pallas as pl
from jax.experimental.pallas import tpu as pltpu

_MASK_ID = 99999
_PAD_ID = 0
_MAX_STEPS = 64
_DT = (1.0 - 1e-05) / (_MAX_STEPS + 1)

_KS0 = np.uint32(1948878966)
_KS1 = np.uint32(4237131848)
_KS2 = np.uint32(int(_KS0) ^ int(_KS1) ^ 0x1BD11BDA)
_KS = (_KS0, _KS1, _KS2)
_ROT = ((13, 15, 26, 6), (17, 29, 16, 24))

_ROWS = 8
_CH = 2048


def _threefry_bits(n):
    x0 = jnp.full(n.shape, _KS0, dtype=jnp.uint32)
    x1 = n + _KS1
    for step in range(5):
        for r in _ROT[step % 2]:
            x0 = x0 + x1
            x1 = ((x1 << r) | (x1 >> (32 - r))) ^ x0
        x0 = x0 + _KS[(step + 1) % 3]
        x1 = x1 + _KS[(step + 2) % 3] + np.uint32(step + 1)
    return x0 ^ x1


def _tau_step_kernel(coef_ref, xt_ref, vl_ref, rate_ref, newxt_ref, *, V):
    i = pl.program_id(0)
    coef = coef_ref[0]
    xtv = xt_ref[0]
    n_full = (V + _CH - 1) // _CH - 1
    tail = n_full * _CH
    lane0 = jax.lax.broadcasted_iota(jnp.int32, (_ROWS, _CH), 1)
    tlanes = lane0 + tail
    tvalid = tlanes < V

    def max_body(c, m):
        return jnp.maximum(m, vl_ref[0, :, pl.ds(c * _CH, _CH)])

    m = jax.lax.fori_loop(
        0, n_full, max_body, jnp.full((_ROWS, _CH), -jnp.inf, jnp.float32))
    x_t = vl_ref[0, :, pl.ds(tail, _CH)]
    m = jnp.maximum(m, jnp.where(tvalid, x_t, -jnp.inf))
    rowmax = jnp.max(m, axis=1, keepdims=True)

    def sum_body(c, s):
        x_c = vl_ref[0, :, pl.ds(c * _CH, _CH)]
        return s + jnp.exp(x_c - rowmax)

    s = jax.lax.fori_loop(
        0, n_full, sum_body, jnp.zeros((_ROWS, _CH), jnp.float32))
    s = s + jnp.where(tvalid, jnp.exp(x_t - rowmax), 0.0)
    recip = coef / jnp.sum(s, axis=1, keepdims=True)

    subl = jax.lax.broadcasted_iota(jnp.int32, (_ROWS, 1), 0)
    rowV = (i * _ROWS + subl) * V

    def rng_cond(start, r_c):
        n = (lane0 + (rowV + start)).astype(jnp.uint32)
        bits = _threefry_bits(n)
        u = jax.lax.bitcast_convert_type(
            (bits >> 9) | np.uint32(0x3F800000), jnp.float32) - jnp.float32(1.0)
        return jnp.log(u) > -r_c

    def b_body(c, acc):
        start = c * _CH
        x_c = vl_ref[0, :, pl.ds(start, _CH)]
        r_c = jnp.exp(x_c - rowmax) * recip
        rate_ref[0, :, pl.ds(start, _CH)] = r_c
        return acc | rng_cond(start, r_c).astype(jnp.int32)

    acc = jax.lax.fori_loop(
        0, n_full, b_body, jnp.zeros((_ROWS, _CH), jnp.int32))
    r_t = jnp.exp(x_t - rowmax) * recip
    r_t = jnp.where(tlanes == (V - 1), jnp.float32(0.0), r_t)
    rate_ref[0, :, pl.ds(tail, _CH)] = r_t
    acc = acc | (rng_cond(tail, r_t) & tvalid).astype(jnp.int32)
    ev = jnp.max(acc, axis=1, keepdims=True) > 0

    newxt_ref[0] = xtv

    @pl.when(jnp.max(acc) > 0)
    def _reveal():
        def argmax_body(c, best):
            gidx = lane0 + c * _CH
            x_c = vl_ref[0, :, pl.ds(c * _CH, _CH)]
            return jnp.minimum(best, jnp.where(x_c == rowmax, gidx, V))

        best = jax.lax.fori_loop(
            0, n_full, argmax_body, jnp.full((_ROWS, _CH), V, jnp.int32))
        hit_t = (x_t == rowmax) & tvalid
        best = jnp.minimum(best, jnp.where(hit_t, tlanes, V))
        amax = jnp.min(best, axis=1, keepdims=True)
        is_masked = xtv == _MASK_ID
        new = jnp.where(is_masked & ev, amax, xtv)
        newxt_ref[0] = jnp.where(xtv == _PAD_ID, _PAD_ID, new)


@jax.jit
def kernel(vocab_logits, xt, t):
    B, L, V = vocab_logits.shape
    n_rows = B * L
    n_blocks = n_rows // _ROWS
    VP = ((V + _CH - 1) // _CH) * _CH

    vl4 = vocab_logits.reshape(n_blocks, _ROWS, V)
    xt3 = xt.reshape(n_blocks, _ROWS, 1)
    factor = 1.0 / (1.0 - t + 1e-4)
    coef = jnp.float32(_DT) * factor[:, None] * (xt == _MASK_ID)
    coef3 = coef.astype(jnp.float32).reshape(n_blocks, _ROWS, 1)

    rate4, newxt3 = pl.pallas_call(
        functools.partial(_tau_step_kernel, V=V),
        grid=(n_blocks,),
        in_specs=[
            pl.BlockSpec((1, _ROWS, 1), lambda i: (i, 0, 0)),
            pl.BlockSpec((1, _ROWS, 1), lambda i: (i, 0, 0)),
            pl.BlockSpec((1, _ROWS, VP), lambda i: (i, 0, 0)),
        ],
        out_specs=[
            pl.BlockSpec((1, _ROWS, VP), lambda i: (i, 0, 0)),
            pl.BlockSpec((1, _ROWS, 1), lambda i: (i, 0, 0)),
        ],
        out_shape=[
            jax.ShapeDtypeStruct((n_blocks, _ROWS, V), jnp.float32),
            jax.ShapeDtypeStruct((n_blocks, _ROWS, 1), jnp.int32),
        ],
        compiler_params=pltpu.CompilerParams(
            dimension_semantics=("arbitrary",),
        ),
    )(coef3, xt3, vl4)

    return newxt3.reshape(B, L), rate4.reshape(B, L, V)
